# Initial kernel scaffold; baseline (speedup 1.0000x reference)
import jax
import jax.numpy as jnp
from jax.experimental import pallas as pl


def kernel(x, edge_index, input_ids, is_node, mapping, W1l, W1r, b1, W2l, W2r, b2, embed_tokens):
    raise NotImplementedError("write your pallas kernel here")


# speedup vs baseline: 3.0214x; 3.0214x over previous
import jax
import jax.numpy as jnp
from jax import lax
from jax.experimental import pallas as pl
from jax.experimental.pallas import tpu as pltpu
from jax.experimental.pallas import tpu_sc as plsc

NC = 2
NS = 16
NW = NC * NS
LN = 16
D = 128
CH = 128


def _sc_mesh():
    return plsc.VectorSubcoreMesh(
        core_axis_name="c", subcore_axis_name="s", num_cores=NC, num_subcores=NS
    )


def _make_agg(n_chunks, lr, dump, remap_src=False):
    zpt = lr // NS
    nfull, rem = divmod(zpt, CH)

    def body(src_r, dst_r, tab_r, acc_o,
             sidx, didx, lidx, rows, zrow, acc_sh, sem):
        c = lax.axis_index("c")
        s = lax.axis_index("s")
        z16 = jnp.zeros((LN,), jnp.float32)

        def zr(i, carry):
            for j in range(D // LN):
                zrow[i, pl.ds(j * LN, LN)] = z16
            return carry

        lax.fori_loop(0, CH, zr, 0)

        z0 = s * zpt
        for t in range(nfull):
            pltpu.sync_copy(zrow, acc_sh.at[pl.ds(z0 + t * CH, CH)])
        if rem:
            pltpu.sync_copy(zrow.at[pl.ds(0, rem)],
                            acc_sh.at[pl.ds(z0 + nfull * CH, rem)])
        plsc.subcore_barrier()

        pltpu.sync_copy(src_r.at[s], sidx)
        pltpu.sync_copy(dst_r.at[s], didx)

        if remap_src:
            def rm(k, carry):
                for j in range(CH // LN):
                    g16 = sidx[k, pl.ds(j * LN, LN)]
                    sidx[k, pl.ds(j * LN, LN)] = (
                        (g16 & 1) * lr + lax.shift_right_logical(g16, 1))
                return carry

            lax.fori_loop(0, n_chunks, rm, 0)

        def chunk(k, carry):
            pltpu.async_copy(tab_r.at[sidx.at[k]], rows, sem).wait()
            for j in range(CH // LN):
                d16 = didx[k, pl.ds(j * LN, LN)]
                keep = (d16 & 1) == c
                lidx[pl.ds(j * LN, LN)] = jnp.where(
                    keep, lax.shift_right_logical(d16, 1), dump)
            pltpu.sync_copy(rows, acc_sh.at[lidx], add=True)
            return carry

        lax.fori_loop(0, n_chunks, chunk, 0)
        plsc.subcore_barrier()

        pltpu.sync_copy(acc_sh.at[pl.ds(z0, zpt)],
                        acc_o.at[c, pl.ds(z0, zpt)])

    return pl.kernel(
        body,
        out_type=jax.ShapeDtypeStruct((NC, lr, D), jnp.float32),
        mesh=_sc_mesh(),
        scratch_types=[
            pltpu.VMEM((n_chunks, CH), jnp.int32),
            pltpu.VMEM((n_chunks, CH), jnp.int32),
            pltpu.VMEM((CH,), jnp.int32),
            pltpu.VMEM((CH, D), jnp.float32),
            pltpu.VMEM((CH, D), jnp.float32),
            pltpu.VMEM_SHARED((lr, D), jnp.float32),
            pltpu.SemaphoreType.DMA,
        ],
    )


def _make_cnt(n_chunks, lr, dump):
    zpt = lr // NS
    nfull, rem = divmod(zpt, CH)

    def body(dst_r, cnt_o, didx, lidx, ones, zrow, cnt_sh, sem):
        c = lax.axis_index("c")
        s = lax.axis_index("s")
        z16 = jnp.zeros((LN,), jnp.float32)
        o16 = jnp.ones((LN,), jnp.float32)

        def zo(i, carry):
            for j in range(D // LN):
                ones[i, pl.ds(j * LN, LN)] = o16
                zrow[i, pl.ds(j * LN, LN)] = z16
            return carry

        lax.fori_loop(0, CH, zo, 0)

        z0 = s * zpt
        for t in range(nfull):
            pltpu.sync_copy(zrow, cnt_sh.at[pl.ds(z0 + t * CH, CH)])
        if rem:
            pltpu.sync_copy(zrow.at[pl.ds(0, rem)],
                            cnt_sh.at[pl.ds(z0 + nfull * CH, rem)])
        plsc.subcore_barrier()

        pltpu.sync_copy(dst_r.at[s], didx)

        def chunk(k, carry):
            for j in range(CH // LN):
                d16 = didx[k, pl.ds(j * LN, LN)]
                keep = (d16 & 1) == c
                lidx[pl.ds(j * LN, LN)] = jnp.where(
                    keep, lax.shift_right_logical(d16, 1), dump)
            pltpu.sync_copy(ones, cnt_sh.at[lidx], add=True)
            return carry

        lax.fori_loop(0, n_chunks, chunk, 0)
        plsc.subcore_barrier()

        pltpu.sync_copy(cnt_sh.at[pl.ds(z0, zpt)],
                        cnt_o.at[c, pl.ds(z0, zpt)])

    return pl.kernel(
        body,
        out_type=jax.ShapeDtypeStruct((NC, lr, D), jnp.float32),
        mesh=_sc_mesh(),
        scratch_types=[
            pltpu.VMEM((n_chunks, CH), jnp.int32),
            pltpu.VMEM((CH,), jnp.int32),
            pltpu.VMEM((CH, D), jnp.float32),
            pltpu.VMEM((CH, D), jnp.float32),
            pltpu.VMEM_SHARED((lr, D), jnp.float32),
            pltpu.SemaphoreType.DMA,
        ],
    )


def _tc_layer1(a1, cnt_par, xp, w1lt, w1rt, b1, br, nvb):
    lr = xp.shape[1]

    def body(a_ref, c_ref, x_ref, wl_ref, wr_ref, b_ref, r_ref):
        mean = a_ref[0] / jnp.maximum(c_ref[0, :, 0:1], 1.0)
        h = (
            jnp.dot(mean, wl_ref[...], preferred_element_type=jnp.float32)
            + jnp.dot(x_ref[0], wr_ref[...], preferred_element_type=jnp.float32)
            + b_ref[...]
        )
        r_ref[0] = jnp.maximum(h, 0.0)

    return pl.pallas_call(
        body,
        grid=(NC * nvb,),
        in_specs=[
            pl.BlockSpec((1, br, D), lambda i: (i // nvb, i % nvb, 0)),
            pl.BlockSpec((1, br, D), lambda i: (i // nvb, i % nvb, 0)),
            pl.BlockSpec((1, br, D), lambda i: (i // nvb, i % nvb, 0)),
            pl.BlockSpec((D, D), lambda i: (0, 0)),
            pl.BlockSpec((D, D), lambda i: (0, 0)),
            pl.BlockSpec((1, D), lambda i: (0, 0)),
        ],
        out_specs=pl.BlockSpec((1, br, D), lambda i: (i // nvb, i % nvb, 0)),
        out_shape=jax.ShapeDtypeStruct((NC, lr, D), jnp.float32),
    )(a1, cnt_par, xp, w1lt, w1rt, b1)


def _tc_reduce(a2, cnt_par, rp, w2lt, w2rt, b2, n, br, nvb):
    grid = NC * nvb

    def body(a_ref, c_ref, r_ref, wl_ref, wr_ref, b_ref, ne_ref, acc_ref):
        i = pl.program_id(0)

        @pl.when(i == 0)
        def _():
            acc_ref[...] = jnp.zeros((2, D), jnp.float32)

        mean2 = a_ref[0] / jnp.maximum(c_ref[0, :, 0:1], 1.0)
        s2 = jnp.sum(mean2, axis=0)
        sr = jnp.sum(r_ref[0], axis=0)
        acc_ref[...] += jnp.concatenate([s2[None, :], sr[None, :]], axis=0)

        @pl.when(i == grid - 1)
        def _():
            a = acc_ref[...]
            ne_ref[...] = (
                jnp.dot(a[0:1], wl_ref[...], preferred_element_type=jnp.float32)
                + jnp.dot(a[1:2], wr_ref[...],
                          preferred_element_type=jnp.float32)
            ) / float(n) + b_ref[...]

    return pl.pallas_call(
        body,
        grid=(grid,),
        in_specs=[
            pl.BlockSpec((1, br, D), lambda i: (i // nvb, i % nvb, 0)),
            pl.BlockSpec((1, br, D), lambda i: (i // nvb, i % nvb, 0)),
            pl.BlockSpec((1, br, D), lambda i: (i // nvb, i % nvb, 0)),
            pl.BlockSpec((D, D), lambda i: (0, 0)),
            pl.BlockSpec((D, D), lambda i: (0, 0)),
            pl.BlockSpec((1, D), lambda i: (0, 0)),
        ],
        out_specs=pl.BlockSpec((1, D), lambda i: (0, 0)),
        out_shape=jax.ShapeDtypeStruct((1, D), jnp.float32),
        scratch_shapes=[pltpu.VMEM((2, D), jnp.float32)],
    )(a2, cnt_par, rp, w2lt, w2rt, b2)


def _make_embed(seq):
    spw = seq // NW

    def body(emb_r, ids_r, msk_r, ne_r, out_o, idsv, mskv, nev, rows, sem):
        c = lax.axis_index("c")
        s = lax.axis_index("s")
        wid = c * NS + s
        pltpu.sync_copy(ids_r.at[wid], idsv)
        pltpu.sync_copy(msk_r.at[wid], mskv)
        pltpu.sync_copy(ne_r.at[0], nev)
        pltpu.async_copy(emb_r.at[idsv], rows, sem).wait()

        for g in range(spw // LN):
            mv = mskv[pl.ds(g * LN, LN)]
            for l in range(LN):
                e = g * LN + l

                @pl.when(mv[l] != 0)
                def _():
                    for j in range(D // LN):
                        rows[e, pl.ds(j * LN, LN)] = nev[pl.ds(j * LN, LN)]

        pltpu.sync_copy(rows, out_o.at[pl.ds(wid * spw, spw)])

    return pl.kernel(
        body,
        out_type=jax.ShapeDtypeStruct((seq, D), jnp.float32),
        mesh=_sc_mesh(),
        scratch_types=[
            pltpu.VMEM((spw,), jnp.int32),
            pltpu.VMEM((spw,), jnp.int32),
            pltpu.VMEM((D,), jnp.float32),
            pltpu.VMEM((spw, D), jnp.float32),
            pltpu.SemaphoreType.DMA,
        ],
    )


def kernel(x, edge_index, input_ids, is_node, mapping, W1l, W1r, b1, W2l, W2r,
           b2, embed_tokens):
    n, d = x.shape
    assert d == D and n % 2 == 0
    e = edge_index.shape[1]
    batch, seq = input_ids.shape
    hn = n // 2

    src = edge_index[0].astype(jnp.int32)
    dst = edge_index[1].astype(jnp.int32)

    def pad3(a, fill, nw):
        nch = -(-e // (nw * CH))
        ep = nw * nch * CH
        return jnp.concatenate(
            [a, jnp.full((ep - e,), fill, jnp.int32)]).reshape(nw, nch, CH)

    srcA, dstA = pad3(src, 0, NS), pad3(dst, n, NS)

    lr = -(-(hn + 1) // (NS * 8)) * (NS * 8)
    nch = srcA.shape[1]

    a1 = _make_agg(nch, lr, hn)(srcA, dstA, x)
    cnt_par = _make_cnt(nch, lr, hn)(dstA)

    xp = jnp.zeros((NC, lr, D), x.dtype).at[:, :hn].set(
        x.reshape(hn, NC, D).transpose(1, 0, 2))

    br = 200
    nvb = hn // br
    rp = _tc_layer1(a1, cnt_par, xp, W1l.T, W1r.T, b1.reshape(1, D), br, nvb)
    a2 = _make_agg(nch, lr, hn, remap_src=True)(
        srcA, dstA, rp.reshape(NC * lr, D))
    ne = _tc_reduce(a2, cnt_par, rp, W2l.T, W2r.T, b2.reshape(1, D), n, br, nvb)

    ids2 = input_ids.reshape(NW, (batch * seq) // NW).astype(jnp.int32)
    msk2 = is_node.reshape(NW, (batch * seq) // NW).astype(jnp.int32)
    out = _make_embed(batch * seq)(embed_tokens, ids2, msk2, ne)
    return out.reshape(batch, seq, D)


# speedup vs baseline: 3.4690x; 1.1481x over previous
import jax
import jax.numpy as jnp
from jax import lax
from jax.experimental import pallas as pl
from jax.experimental.pallas import tpu as pltpu
from jax.experimental.pallas import tpu_sc as plsc

NC = 2
NS = 16
NW = NC * NS
LN = 16
D = 128
CH = 128


def _sc_mesh():
    return plsc.VectorSubcoreMesh(
        core_axis_name="c", subcore_axis_name="s", num_cores=NC, num_subcores=NS
    )


def _make_agg(n_chunks, lr, dump, remap_src=False):
    zpt = lr // NS
    ZB = 64
    nfull, rem = divmod(zpt, ZB)

    def body(src_r, dst_r, tab_r, acc_o,
             sidx, didx, lidx, rows0, rows1, zrow, acc_sh, sem0, sem1):
        c = lax.axis_index("c")
        s = lax.axis_index("s")
        z16 = jnp.zeros((LN,), jnp.float32)

        def zr(i, carry):
            for j in range(D // LN):
                zrow[i, pl.ds(j * LN, LN)] = z16
            return carry

        lax.fori_loop(0, ZB, zr, 0)

        z0 = s * zpt
        for t in range(nfull):
            pltpu.sync_copy(zrow, acc_sh.at[pl.ds(z0 + t * ZB, ZB)])
        if rem:
            pltpu.sync_copy(zrow.at[pl.ds(0, rem)],
                            acc_sh.at[pl.ds(z0 + nfull * ZB, rem)])
        plsc.subcore_barrier()

        pltpu.sync_copy(src_r.at[s], sidx)
        pltpu.sync_copy(dst_r.at[s], didx)

        if remap_src:
            def rm(k, carry):
                for j in range(CH // LN):
                    g16 = sidx[k, pl.ds(j * LN, LN)]
                    sidx[k, pl.ds(j * LN, LN)] = (
                        (g16 & 1) * lr + lax.shift_right_logical(g16, 1))
                return carry

            lax.fori_loop(0, n_chunks, rm, 0)

        def remap_scatter(k, rows):
            for j in range(CH // LN):
                d16 = didx[k, pl.ds(j * LN, LN)]
                keep = (d16 & 1) == c
                lidx[pl.ds(j * LN, LN)] = jnp.where(
                    keep, lax.shift_right_logical(d16, 1), dump)
            pltpu.sync_copy(rows, acc_sh.at[lidx], add=True)

        pltpu.async_copy(tab_r.at[sidx.at[0]], rows0, sem0)

        def step(t, carry):
            k0 = 2 * t
            k1 = k0 + 1

            @pl.when(k1 < n_chunks)
            def _():
                pltpu.async_copy(tab_r.at[sidx.at[k1]], rows1, sem1)

            pltpu.make_async_copy(tab_r.at[sidx.at[k0]], rows0, sem0).wait()
            remap_scatter(k0, rows0)

            @pl.when(k0 + 2 < n_chunks)
            def _():
                pltpu.async_copy(tab_r.at[sidx.at[k0 + 2]], rows0, sem0)

            @pl.when(k1 < n_chunks)
            def _():
                pltpu.make_async_copy(
                    tab_r.at[sidx.at[k1]], rows1, sem1).wait()
                remap_scatter(k1, rows1)

            return carry

        lax.fori_loop(0, (n_chunks + 1) // 2, step, 0)
        plsc.subcore_barrier()

        pltpu.sync_copy(acc_sh.at[pl.ds(z0, zpt)],
                        acc_o.at[c, pl.ds(z0, zpt)])

    return pl.kernel(
        body,
        out_type=jax.ShapeDtypeStruct((NC, lr, D), jnp.float32),
        mesh=_sc_mesh(),
        scratch_types=[
            pltpu.VMEM((n_chunks, CH), jnp.int32),
            pltpu.VMEM((n_chunks, CH), jnp.int32),
            pltpu.VMEM((CH,), jnp.int32),
            pltpu.VMEM((CH, D), jnp.float32),
            pltpu.VMEM((CH, D), jnp.float32),
            pltpu.VMEM((ZB, D), jnp.float32),
            pltpu.VMEM_SHARED((lr, D), jnp.float32),
            pltpu.SemaphoreType.DMA,
            pltpu.SemaphoreType.DMA,
        ],
    )


def _make_cnt(n_chunks, lr, dump):
    zpt = lr // NS
    ZB = 64
    nfull, rem = divmod(zpt, ZB)

    def body(dst_r, cnt_o, didx, lidx, ones, zrow, cnt_sh, sem):
        c = lax.axis_index("c")
        s = lax.axis_index("s")
        z16 = jnp.zeros((LN,), jnp.float32)
        o16 = jnp.ones((LN,), jnp.float32)

        def zo(i, carry):
            for j in range(D // LN):
                ones[i, pl.ds(j * LN, LN)] = o16
            return carry

        lax.fori_loop(0, CH, zo, 0)

        def zr(i, carry):
            for j in range(D // LN):
                zrow[i, pl.ds(j * LN, LN)] = z16
            return carry

        lax.fori_loop(0, ZB, zr, 0)

        z0 = s * zpt
        for t in range(nfull):
            pltpu.sync_copy(zrow, cnt_sh.at[pl.ds(z0 + t * ZB, ZB)])
        if rem:
            pltpu.sync_copy(zrow.at[pl.ds(0, rem)],
                            cnt_sh.at[pl.ds(z0 + nfull * ZB, rem)])
        plsc.subcore_barrier()

        pltpu.sync_copy(dst_r.at[s], didx)

        def chunk(k, carry):
            for j in range(CH // LN):
                d16 = didx[k, pl.ds(j * LN, LN)]
                keep = (d16 & 1) == c
                lidx[pl.ds(j * LN, LN)] = jnp.where(
                    keep, lax.shift_right_logical(d16, 1), dump)
            pltpu.sync_copy(ones, cnt_sh.at[lidx], add=True)
            return carry

        lax.fori_loop(0, n_chunks, chunk, 0)
        plsc.subcore_barrier()

        pltpu.sync_copy(cnt_sh.at[pl.ds(z0, zpt)],
                        cnt_o.at[c, pl.ds(z0, zpt)])

    return pl.kernel(
        body,
        out_type=jax.ShapeDtypeStruct((NC, lr, D), jnp.float32),
        mesh=_sc_mesh(),
        scratch_types=[
            pltpu.VMEM((n_chunks, CH), jnp.int32),
            pltpu.VMEM((CH,), jnp.int32),
            pltpu.VMEM((CH, D), jnp.float32),
            pltpu.VMEM((ZB, D), jnp.float32),
            pltpu.VMEM_SHARED((lr, D), jnp.float32),
            pltpu.SemaphoreType.DMA,
        ],
    )


def _tc_layer1(a1, cnt_par, xp, w1lt, w1rt, b1, br, nvb):
    lr = xp.shape[1]

    def body(a_ref, c_ref, x_ref, wl_ref, wr_ref, b_ref, r_ref):
        mean = a_ref[0] / jnp.maximum(c_ref[0, :, 0:1], 1.0)
        h = (
            jnp.dot(mean, wl_ref[...], preferred_element_type=jnp.float32)
            + jnp.dot(x_ref[0], wr_ref[...], preferred_element_type=jnp.float32)
            + b_ref[...]
        )
        r_ref[0] = jnp.maximum(h, 0.0)

    return pl.pallas_call(
        body,
        grid=(NC * nvb,),
        in_specs=[
            pl.BlockSpec((1, br, D), lambda i: (i // nvb, i % nvb, 0)),
            pl.BlockSpec((1, br, D), lambda i: (i // nvb, i % nvb, 0)),
            pl.BlockSpec((1, br, D), lambda i: (i // nvb, i % nvb, 0)),
            pl.BlockSpec((D, D), lambda i: (0, 0)),
            pl.BlockSpec((D, D), lambda i: (0, 0)),
            pl.BlockSpec((1, D), lambda i: (0, 0)),
        ],
        out_specs=pl.BlockSpec((1, br, D), lambda i: (i // nvb, i % nvb, 0)),
        out_shape=jax.ShapeDtypeStruct((NC, lr, D), jnp.float32),
    )(a1, cnt_par, xp, w1lt, w1rt, b1)


def _tc_reduce(a2, cnt_par, rp, w2lt, w2rt, b2, n, br, nvb):
    grid = NC * nvb

    def body(a_ref, c_ref, r_ref, wl_ref, wr_ref, b_ref, ne_ref, acc_ref):
        i = pl.program_id(0)

        @pl.when(i == 0)
        def _():
            acc_ref[...] = jnp.zeros((2, D), jnp.float32)

        mean2 = a_ref[0] / jnp.maximum(c_ref[0, :, 0:1], 1.0)
        s2 = jnp.sum(mean2, axis=0)
        sr = jnp.sum(r_ref[0], axis=0)
        acc_ref[...] += jnp.concatenate([s2[None, :], sr[None, :]], axis=0)

        @pl.when(i == grid - 1)
        def _():
            a = acc_ref[...]
            ne_ref[...] = (
                jnp.dot(a[0:1], wl_ref[...], preferred_element_type=jnp.float32)
                + jnp.dot(a[1:2], wr_ref[...],
                          preferred_element_type=jnp.float32)
            ) / float(n) + b_ref[...]

    return pl.pallas_call(
        body,
        grid=(grid,),
        in_specs=[
            pl.BlockSpec((1, br, D), lambda i: (i // nvb, i % nvb, 0)),
            pl.BlockSpec((1, br, D), lambda i: (i // nvb, i % nvb, 0)),
            pl.BlockSpec((1, br, D), lambda i: (i // nvb, i % nvb, 0)),
            pl.BlockSpec((D, D), lambda i: (0, 0)),
            pl.BlockSpec((D, D), lambda i: (0, 0)),
            pl.BlockSpec((1, D), lambda i: (0, 0)),
        ],
        out_specs=pl.BlockSpec((1, D), lambda i: (0, 0)),
        out_shape=jax.ShapeDtypeStruct((1, D), jnp.float32),
        scratch_shapes=[pltpu.VMEM((2, D), jnp.float32)],
    )(a2, cnt_par, rp, w2lt, w2rt, b2)


def _make_embed(seq):
    spw = seq // NW

    def body(emb_r, ids_r, msk_r, ne_r, out_o, idsv, mskv, nev, rows, sem):
        c = lax.axis_index("c")
        s = lax.axis_index("s")
        wid = c * NS + s
        pltpu.sync_copy(ids_r.at[wid], idsv)
        pltpu.sync_copy(msk_r.at[wid], mskv)
        pltpu.sync_copy(ne_r.at[0], nev)
        pltpu.async_copy(emb_r.at[idsv], rows, sem).wait()

        for g in range(spw // LN):
            mv = mskv[pl.ds(g * LN, LN)]
            for l in range(LN):
                e = g * LN + l

                @pl.when(mv[l] != 0)
                def _():
                    for j in range(D // LN):
                        rows[e, pl.ds(j * LN, LN)] = nev[pl.ds(j * LN, LN)]

        pltpu.sync_copy(rows, out_o.at[pl.ds(wid * spw, spw)])

    return pl.kernel(
        body,
        out_type=jax.ShapeDtypeStruct((seq, D), jnp.float32),
        mesh=_sc_mesh(),
        scratch_types=[
            pltpu.VMEM((spw,), jnp.int32),
            pltpu.VMEM((spw,), jnp.int32),
            pltpu.VMEM((D,), jnp.float32),
            pltpu.VMEM((spw, D), jnp.float32),
            pltpu.SemaphoreType.DMA,
        ],
    )


def kernel(x, edge_index, input_ids, is_node, mapping, W1l, W1r, b1, W2l, W2r,
           b2, embed_tokens):
    n, d = x.shape
    assert d == D and n % 2 == 0
    e = edge_index.shape[1]
    batch, seq = input_ids.shape
    hn = n // 2

    src = edge_index[0].astype(jnp.int32)
    dst = edge_index[1].astype(jnp.int32)

    def pad3(a, fill, nw):
        nch = -(-e // (nw * CH))
        ep = nw * nch * CH
        return jnp.concatenate(
            [a, jnp.full((ep - e,), fill, jnp.int32)]).reshape(nw, nch, CH)

    srcA, dstA = pad3(src, 0, NS), pad3(dst, n, NS)

    lr = -(-(hn + 1) // (NS * 8)) * (NS * 8)
    nch = srcA.shape[1]

    a1 = _make_agg(nch, lr, hn)(srcA, dstA, x)
    cnt_par = _make_cnt(nch, lr, hn)(dstA)

    xp = jnp.zeros((NC, lr, D), x.dtype).at[:, :hn].set(
        x.reshape(hn, NC, D).transpose(1, 0, 2))

    br = 200
    nvb = hn // br
    rp = _tc_layer1(a1, cnt_par, xp, W1l.T, W1r.T, b1.reshape(1, D), br, nvb)
    a2 = _make_agg(nch, lr, hn, remap_src=True)(
        srcA, dstA, rp.reshape(NC * lr, D))
    ne = _tc_reduce(a2, cnt_par, rp, W2l.T, W2r.T, b2.reshape(1, D), n, br, nvb)

    ids2 = input_ids.reshape(NW, (batch * seq) // NW).astype(jnp.int32)
    msk2 = is_node.reshape(NW, (batch * seq) // NW).astype(jnp.int32)
    out = _make_embed(batch * seq)(embed_tokens, ids2, msk2, ne)
    return out.reshape(batch, seq, D)
